# baseline (speedup 1.0000x reference)
import functools

import jax
import jax.numpy as jnp
from jax.experimental import pallas as pl
from jax.experimental.pallas import tpu as pltpu


def _build_adj_kernel(src_ref, dst_ref, adj_ref, *, n_chunk, n_words):
    step = pl.program_id(0)

    @pl.when(step == 0)
    def _():
        adj_ref[...] = jnp.zeros_like(adj_ref)

    iota = jax.lax.broadcasted_iota(jnp.int32, (1, n_words), 1)
    one = jnp.int32(1)

    def body(i, carry):
        u = src_ref[0, 0, i]
        v = dst_ref[0, 0, i]
        mv = jnp.where(iota == (v >> 5), one << (v & 31), 0)
        adj_ref[pl.ds(u, 1), :] = adj_ref[pl.ds(u, 1), :] | mv
        mu = jnp.where(iota == (u >> 5), one << (u & 31), 0)
        adj_ref[pl.ds(v, 1), :] = adj_ref[pl.ds(v, 1), :] | mu
        return carry

    jax.lax.fori_loop(0, n_chunk, body, 0)


def _popcount(x):
    x = x - ((x >> 1) & 0x55555555)
    x = (x & 0x33333333) + ((x >> 2) & 0x33333333)
    x = (x + (x >> 4)) & 0x0F0F0F0F
    return (x * 0x01010101) >> 24


def _norm_kernel(src_ref, dst_ref, adj_ref, norm_ref, *, n_chunk):
    def body(i, carry):
        u = src_ref[0, 0, i]
        v = dst_ref[0, 0, i]
        ru = adj_ref[pl.ds(u, 1), :]
        rv = adj_ref[pl.ds(v, 1), :]
        inter = jnp.sum(_popcount(ru & rv)).astype(jnp.float32)
        union = jnp.sum(_popcount(ru | rv)).astype(jnp.float32)
        norm_ref[pl.ds(i, 1), :] = (inter / union).reshape(1, 1)
        return carry

    jax.lax.fori_loop(0, n_chunk, body, 0)


def _matmul_kernel(x_ref, w_ref, o_ref):
    o_ref[...] = jnp.dot(x_ref[...], w_ref[...],
                         preferred_element_type=jnp.float32)


def _agg_kernel(src_ref, dst_ref, norm_ref, hl_ref, b_ref, out_ref, agg_ref,
                *, n_chunk):
    step = pl.program_id(0)
    nsteps = pl.num_programs(0)

    @pl.when(step == 0)
    def _():
        agg_ref[...] = jnp.full_like(agg_ref, -jnp.inf)

    def body(i, carry):
        u = src_ref[0, 0, i]
        v = dst_ref[0, 0, i]
        nrm = norm_ref[pl.ds(i, 1), :]
        msg = hl_ref[pl.ds(u, 1), :] * nrm
        agg_ref[pl.ds(v, 1), :] = jnp.maximum(agg_ref[pl.ds(v, 1), :], msg)
        return carry

    jax.lax.fori_loop(0, n_chunk, body, 0)

    @pl.when(step == nsteps - 1)
    def _():
        out_ref[...] = jnp.maximum(agg_ref[...] + b_ref[...], 0.0)


def _head_kernel(h_ref, w_ref, b_ref, o_ref):
    z = jnp.dot(h_ref[...], w_ref[...],
                preferred_element_type=jnp.float32) + b_ref[...]
    m = jnp.max(z, axis=-1, keepdims=True)
    lse = m + jnp.log(jnp.sum(jnp.exp(z - m), axis=-1, keepdims=True))
    o_ref[...] = z - lse


def kernel(x, edge_index, W0, b0, W1, b1, Wl, bl):
    n, d_in = x.shape
    d_hid = W0.shape[1]
    n_cls = Wl.shape[1]
    e = edge_index.shape[1]

    loops = jnp.arange(n, dtype=edge_index.dtype)
    ei = jnp.concatenate([edge_index, jnp.stack([loops, loops])], axis=1)
    e_tot = e + n
    ch = 4096
    nb = -(-e_tot // ch)
    e_pad = nb * ch
    ei = jnp.pad(ei, ((0, 0), (0, e_pad - e_tot)))
    src3 = ei[0].reshape(nb, 1, ch)
    dst3 = ei[1].reshape(nb, 1, ch)

    n_words = (-(-n // 32) + 127) // 128 * 128

    idx_spec = pl.BlockSpec((1, 1, ch), lambda i: (i, 0, 0),
                            memory_space=pltpu.SMEM)

    adj = pl.pallas_call(
        functools.partial(_build_adj_kernel, n_chunk=ch, n_words=n_words),
        grid=(nb,),
        in_specs=[idx_spec, idx_spec],
        out_specs=pl.BlockSpec((n, n_words), lambda i: (0, 0)),
        out_shape=jax.ShapeDtypeStruct((n, n_words), jnp.int32),
    )(src3, dst3)

    norm = pl.pallas_call(
        functools.partial(_norm_kernel, n_chunk=ch),
        grid=(nb,),
        in_specs=[idx_spec, idx_spec,
                  pl.BlockSpec((n, n_words), lambda i: (0, 0))],
        out_specs=pl.BlockSpec((ch, 1), lambda i: (i, 0)),
        out_shape=jax.ShapeDtypeStruct((e_pad, 1), jnp.float32),
    )(src3, dst3, adj)

    row_blk = 1000 if n % 1000 == 0 else n
    n_row_blks = n // row_blk

    def matmul(h, w):
        d_out = w.shape[1]
        return pl.pallas_call(
            _matmul_kernel,
            grid=(n_row_blks,),
            in_specs=[pl.BlockSpec((row_blk, w.shape[0]), lambda i: (i, 0)),
                      pl.BlockSpec(w.shape, lambda i: (0, 0))],
            out_specs=pl.BlockSpec((row_blk, d_out), lambda i: (i, 0)),
            out_shape=jax.ShapeDtypeStruct((n, d_out), jnp.float32),
        )(h, w)

    def layer(h, w, b):
        hl = matmul(h, w)
        return pl.pallas_call(
            functools.partial(_agg_kernel, n_chunk=ch),
            grid=(nb,),
            in_specs=[idx_spec, idx_spec,
                      pl.BlockSpec((ch, 1), lambda i: (i, 0)),
                      pl.BlockSpec((n, d_hid), lambda i: (0, 0)),
                      pl.BlockSpec((1, d_hid), lambda i: (0, 0))],
            out_specs=pl.BlockSpec((n, d_hid), lambda i: (0, 0)),
            out_shape=jax.ShapeDtypeStruct((n, d_hid), jnp.float32),
            scratch_shapes=[pltpu.VMEM((n, d_hid), jnp.float32)],
        )(src3, dst3, norm, hl, b.reshape(1, d_hid))

    h = layer(x, W0, b0)
    h = layer(h, W1, b1)

    return pl.pallas_call(
        _head_kernel,
        grid=(n_row_blks,),
        in_specs=[pl.BlockSpec((row_blk, d_hid), lambda i: (i, 0)),
                  pl.BlockSpec((d_hid, n_cls), lambda i: (0, 0)),
                  pl.BlockSpec((1, n_cls), lambda i: (0, 0))],
        out_specs=pl.BlockSpec((row_blk, n_cls), lambda i: (i, 0)),
        out_shape=jax.ShapeDtypeStruct((n, n_cls), jnp.float32),
    )(h, Wl, bl.reshape(1, n_cls))


# speedup vs baseline: 8.3806x; 8.3806x over previous
import functools

import jax
import jax.numpy as jnp
from jax.experimental import pallas as pl
from jax.experimental.pallas import tpu as pltpu


def _build_adj_kernel(src_ref, dst_ref, adj_ref, *, n_chunk, n_words):
    step = pl.program_id(0)

    @pl.when(step == 0)
    def _():
        adj_ref[...] = jnp.zeros_like(adj_ref)

    iota = jax.lax.broadcasted_iota(jnp.int32, (1, n_words), 1)
    one = jnp.int32(1)

    def body(i, carry):
        u = src_ref[0, 0, i]
        v = dst_ref[0, 0, i]
        mv = jnp.where(iota == (v >> 5), one << (v & 31), 0)
        adj_ref[pl.ds(u, 1), :] = adj_ref[pl.ds(u, 1), :] | mv
        mu = jnp.where(iota == (u >> 5), one << (u & 31), 0)
        adj_ref[pl.ds(v, 1), :] = adj_ref[pl.ds(v, 1), :] | mu
        return carry

    jax.lax.fori_loop(0, n_chunk, body, 0, unroll=4)


def _popcount(x):
    x = x - ((x >> 1) & 0x55555555)
    x = (x & 0x33333333) + ((x >> 2) & 0x33333333)
    x = (x + (x >> 4)) & 0x0F0F0F0F
    return (x * 0x01010101) >> 24


def _norm_kernel(src_ref, dst_ref, adj_ref, norm_ref, *, n_chunk):
    def body(j, carry):
        i8 = j * 8
        ru = jnp.concatenate(
            [adj_ref[pl.ds(src_ref[0, 0, i8 + k], 1), :] for k in range(8)],
            axis=0)
        rv = jnp.concatenate(
            [adj_ref[pl.ds(dst_ref[0, 0, i8 + k], 1), :] for k in range(8)],
            axis=0)
        inter = jnp.sum(_popcount(ru & rv), axis=1,
                        keepdims=True).astype(jnp.float32)
        union = jnp.sum(_popcount(ru | rv), axis=1,
                        keepdims=True).astype(jnp.float32)
        norm_ref[pl.ds(i8, 8), :] = inter / union
        return carry

    jax.lax.fori_loop(0, n_chunk // 8, body, 0, unroll=2)


def _matmul_kernel(x_ref, w_ref, o_ref):
    o_ref[...] = jnp.dot(x_ref[...], w_ref[...],
                         preferred_element_type=jnp.float32)


def _agg_kernel(src_ref, dst_ref, norm_ref, hl_ref, b_ref, out_ref, agg_ref,
                *, n_chunk):
    step = pl.program_id(0)
    nsteps = pl.num_programs(0)

    @pl.when(step == 0)
    def _():
        agg_ref[...] = jnp.full_like(agg_ref, -jnp.inf)

    def body(j, carry):
        i8 = j * 8
        rows = jnp.concatenate(
            [hl_ref[pl.ds(src_ref[0, 0, i8 + k], 1), :] for k in range(8)],
            axis=0)
        msgs = rows * norm_ref[pl.ds(i8, 8), :]
        for k in range(8):
            v = dst_ref[0, 0, i8 + k]
            agg_ref[pl.ds(v, 1), :] = jnp.maximum(agg_ref[pl.ds(v, 1), :],
                                                  msgs[k:k + 1, :])
        return carry

    jax.lax.fori_loop(0, n_chunk // 8, body, 0)

    @pl.when(step == nsteps - 1)
    def _():
        out_ref[...] = jnp.maximum(agg_ref[...] + b_ref[...], 0.0)


def _head_kernel(h_ref, w_ref, b_ref, o_ref):
    z = jnp.dot(h_ref[...], w_ref[...],
                preferred_element_type=jnp.float32) + b_ref[...]
    m = jnp.max(z, axis=-1, keepdims=True)
    lse = m + jnp.log(jnp.sum(jnp.exp(z - m), axis=-1, keepdims=True))
    o_ref[...] = z - lse


def kernel(x, edge_index, W0, b0, W1, b1, Wl, bl):
    n, d_in = x.shape
    d_hid = W0.shape[1]
    n_cls = Wl.shape[1]
    e = edge_index.shape[1]

    loops = jnp.arange(n, dtype=edge_index.dtype)
    ei = jnp.concatenate([edge_index, jnp.stack([loops, loops])], axis=1)
    e_tot = e + n
    ch = 4096
    nb = -(-e_tot // ch)
    e_pad = nb * ch
    ei = jnp.pad(ei, ((0, 0), (0, e_pad - e_tot)))
    src3 = ei[0].reshape(nb, 1, ch)
    dst3 = ei[1].reshape(nb, 1, ch)

    n_words = (-(-n // 32) + 127) // 128 * 128

    idx_spec = pl.BlockSpec((1, 1, ch), lambda i: (i, 0, 0),
                            memory_space=pltpu.SMEM)

    adj = pl.pallas_call(
        functools.partial(_build_adj_kernel, n_chunk=ch, n_words=n_words),
        grid=(nb,),
        in_specs=[idx_spec, idx_spec],
        out_specs=pl.BlockSpec((n, n_words), lambda i: (0, 0)),
        out_shape=jax.ShapeDtypeStruct((n, n_words), jnp.int32),
    )(src3, dst3)

    norm = pl.pallas_call(
        functools.partial(_norm_kernel, n_chunk=ch),
        grid=(nb,),
        in_specs=[idx_spec, idx_spec,
                  pl.BlockSpec((n, n_words), lambda i: (0, 0))],
        out_specs=pl.BlockSpec((ch, 1), lambda i: (i, 0)),
        out_shape=jax.ShapeDtypeStruct((e_pad, 1), jnp.float32),
    )(src3, dst3, adj)

    row_blk = 1000 if n % 1000 == 0 else n
    n_row_blks = n // row_blk

    def matmul(h, w):
        d_out = w.shape[1]
        return pl.pallas_call(
            _matmul_kernel,
            grid=(n_row_blks,),
            in_specs=[pl.BlockSpec((row_blk, w.shape[0]), lambda i: (i, 0)),
                      pl.BlockSpec(w.shape, lambda i: (0, 0))],
            out_specs=pl.BlockSpec((row_blk, d_out), lambda i: (i, 0)),
            out_shape=jax.ShapeDtypeStruct((n, d_out), jnp.float32),
        )(h, w)

    def layer(h, w, b):
        hl = matmul(h, w)
        return pl.pallas_call(
            functools.partial(_agg_kernel, n_chunk=ch),
            grid=(nb,),
            in_specs=[idx_spec, idx_spec,
                      pl.BlockSpec((ch, 1), lambda i: (i, 0)),
                      pl.BlockSpec((n, d_hid), lambda i: (0, 0)),
                      pl.BlockSpec((1, d_hid), lambda i: (0, 0))],
            out_specs=pl.BlockSpec((n, d_hid), lambda i: (0, 0)),
            out_shape=jax.ShapeDtypeStruct((n, d_hid), jnp.float32),
            scratch_shapes=[pltpu.VMEM((n, d_hid), jnp.float32)],
        )(src3, dst3, norm, hl, b.reshape(1, d_hid))

    h = layer(x, W0, b0)
    h = layer(h, W1, b1)

    return pl.pallas_call(
        _head_kernel,
        grid=(n_row_blks,),
        in_specs=[pl.BlockSpec((row_blk, d_hid), lambda i: (i, 0)),
                  pl.BlockSpec((d_hid, n_cls), lambda i: (0, 0)),
                  pl.BlockSpec((1, n_cls), lambda i: (0, 0))],
        out_specs=pl.BlockSpec((row_blk, n_cls), lambda i: (i, 0)),
        out_shape=jax.ShapeDtypeStruct((n, n_cls), jnp.float32),
    )(h, Wl, bl.reshape(1, n_cls))


# speedup vs baseline: 10.1533x; 1.2115x over previous
import functools

import jax
import jax.numpy as jnp
from jax.experimental import pallas as pl
from jax.experimental.pallas import tpu as pltpu


def _build_adj_kernel(src_ref, dst_ref, adj_ref, *, n_chunk, n_words):
    step = pl.program_id(0)

    @pl.when(step == 0)
    def _():
        adj_ref[...] = jnp.zeros_like(adj_ref)

    iota = jax.lax.broadcasted_iota(jnp.int32, (1, n_words), 1)
    one = jnp.int32(1)

    def body(j, carry):
        i4 = j * 4
        uv = [(src_ref[0, 0, i4 + k], dst_ref[0, 0, i4 + k])
              for k in range(4)]
        masks = [(u, jnp.where(iota == (v >> 5), one << (v & 31), 0))
                 for (u, v) in uv]
        masks += [(v, jnp.where(iota == (u >> 5), one << (u & 31), 0))
                  for (u, v) in uv]
        for (r, m) in masks:
            adj_ref[pl.ds(r, 1), :] = adj_ref[pl.ds(r, 1), :] | m
        return carry

    jax.lax.fori_loop(0, n_chunk // 4, body, 0, unroll=2)


def _popcount(x):
    x = x - ((x >> 1) & 0x55555555)
    x = (x & 0x33333333) + ((x >> 2) & 0x33333333)
    x = (x + (x >> 4)) & 0x0F0F0F0F
    return (x * 0x01010101) >> 24


def _norm_kernel(src_ref, dst_ref, adj_ref, norm_ref, *, n_chunk):
    def body(j, carry):
        i8 = j * 8
        ru = jnp.concatenate(
            [adj_ref[pl.ds(src_ref[0, 0, i8 + k], 1), :] for k in range(8)],
            axis=0)
        rv = jnp.concatenate(
            [adj_ref[pl.ds(dst_ref[0, 0, i8 + k], 1), :] for k in range(8)],
            axis=0)
        inter = jnp.sum(_popcount(ru & rv), axis=1,
                        keepdims=True).astype(jnp.float32)
        union = jnp.sum(_popcount(ru | rv), axis=1,
                        keepdims=True).astype(jnp.float32)
        norm_ref[pl.ds(i8, 8), :] = inter / union
        return carry

    jax.lax.fori_loop(0, n_chunk // 8, body, 0, unroll=2)


def _matmul_kernel(x_ref, w_ref, o_ref):
    o_ref[...] = jnp.dot(x_ref[...], w_ref[...],
                         preferred_element_type=jnp.float32)


def _agg_kernel(src_ref, dst_ref, norm_ref, hl_ref, b_ref, out_ref, agg_ref,
                *, n_chunk):
    step = pl.program_id(0)
    nsteps = pl.num_programs(0)

    @pl.when(step == 0)
    def _():
        agg_ref[...] = jnp.full_like(agg_ref, -jnp.inf)

    def body(j, carry):
        i8 = j * 8
        rows = jnp.concatenate(
            [hl_ref[pl.ds(src_ref[0, 0, i8 + k], 1), :] for k in range(8)],
            axis=0)
        msgs = rows * norm_ref[pl.ds(i8, 8), :]
        for k in range(8):
            v = dst_ref[0, 0, i8 + k]
            agg_ref[pl.ds(v, 1), :] = jnp.maximum(agg_ref[pl.ds(v, 1), :],
                                                  msgs[k:k + 1, :])
        return carry

    jax.lax.fori_loop(0, n_chunk // 8, body, 0, unroll=2)

    @pl.when(step == nsteps - 1)
    def _():
        out_ref[...] = jnp.maximum(agg_ref[...] + b_ref[...], 0.0)


def _head_kernel(h_ref, w_ref, b_ref, o_ref):
    z = jnp.dot(h_ref[...], w_ref[...],
                preferred_element_type=jnp.float32) + b_ref[...]
    m = jnp.max(z, axis=-1, keepdims=True)
    lse = m + jnp.log(jnp.sum(jnp.exp(z - m), axis=-1, keepdims=True))
    o_ref[...] = z - lse


def kernel(x, edge_index, W0, b0, W1, b1, Wl, bl):
    n, d_in = x.shape
    d_hid = W0.shape[1]
    n_cls = Wl.shape[1]
    e = edge_index.shape[1]

    loops = jnp.arange(n, dtype=edge_index.dtype)
    ei = jnp.concatenate([edge_index, jnp.stack([loops, loops])], axis=1)
    e_tot = e + n
    ch = 4096
    nb = -(-e_tot // ch)
    e_pad = nb * ch
    ei = jnp.pad(ei, ((0, 0), (0, e_pad - e_tot)))
    src3 = ei[0].reshape(nb, 1, ch)
    dst3 = ei[1].reshape(nb, 1, ch)

    n_words = (-(-n // 32) + 127) // 128 * 128

    idx_spec = pl.BlockSpec((1, 1, ch), lambda i: (i, 0, 0),
                            memory_space=pltpu.SMEM)

    adj = pl.pallas_call(
        functools.partial(_build_adj_kernel, n_chunk=ch, n_words=n_words),
        grid=(nb,),
        in_specs=[idx_spec, idx_spec],
        out_specs=pl.BlockSpec((n, n_words), lambda i: (0, 0)),
        out_shape=jax.ShapeDtypeStruct((n, n_words), jnp.int32),
    )(src3, dst3)

    norm = pl.pallas_call(
        functools.partial(_norm_kernel, n_chunk=ch),
        grid=(nb,),
        in_specs=[idx_spec, idx_spec,
                  pl.BlockSpec((n, n_words), lambda i: (0, 0))],
        out_specs=pl.BlockSpec((ch, 1), lambda i: (i, 0)),
        out_shape=jax.ShapeDtypeStruct((e_pad, 1), jnp.float32),
    )(src3, dst3, adj)

    row_blk = 1000 if n % 1000 == 0 else n
    n_row_blks = n // row_blk

    def matmul(h, w):
        d_out = w.shape[1]
        return pl.pallas_call(
            _matmul_kernel,
            grid=(n_row_blks,),
            in_specs=[pl.BlockSpec((row_blk, w.shape[0]), lambda i: (i, 0)),
                      pl.BlockSpec(w.shape, lambda i: (0, 0))],
            out_specs=pl.BlockSpec((row_blk, d_out), lambda i: (i, 0)),
            out_shape=jax.ShapeDtypeStruct((n, d_out), jnp.float32),
        )(h, w)

    def layer(h, w, b):
        hl = matmul(h, w)
        return pl.pallas_call(
            functools.partial(_agg_kernel, n_chunk=ch),
            grid=(nb,),
            in_specs=[idx_spec, idx_spec,
                      pl.BlockSpec((ch, 1), lambda i: (i, 0)),
                      pl.BlockSpec((n, d_hid), lambda i: (0, 0)),
                      pl.BlockSpec((1, d_hid), lambda i: (0, 0))],
            out_specs=pl.BlockSpec((n, d_hid), lambda i: (0, 0)),
            out_shape=jax.ShapeDtypeStruct((n, d_hid), jnp.float32),
            scratch_shapes=[pltpu.VMEM((n, d_hid), jnp.float32)],
        )(src3, dst3, norm, hl, b.reshape(1, d_hid))

    h = layer(x, W0, b0)
    h = layer(h, W1, b1)

    return pl.pallas_call(
        _head_kernel,
        grid=(n_row_blks,),
        in_specs=[pl.BlockSpec((row_blk, d_hid), lambda i: (i, 0)),
                  pl.BlockSpec((d_hid, n_cls), lambda i: (0, 0)),
                  pl.BlockSpec((1, n_cls), lambda i: (0, 0))],
        out_specs=pl.BlockSpec((row_blk, n_cls), lambda i: (i, 0)),
        out_shape=jax.ShapeDtypeStruct((n, n_cls), jnp.float32),
    )(h, Wl, bl.reshape(1, n_cls))


# speedup vs baseline: 11.7051x; 1.1528x over previous
import functools

import jax
import jax.numpy as jnp
from jax.experimental import pallas as pl
from jax.experimental.pallas import tpu as pltpu


def _build_adj_kernel(src_ref, dst_ref, adj_ref, *, n_chunk, n_words):
    step = pl.program_id(0)

    @pl.when(step == 0)
    def _():
        adj_ref[...] = jnp.zeros_like(adj_ref)

    iota = jax.lax.broadcasted_iota(jnp.int32, (1, n_words), 1)
    one = jnp.int32(1)

    def body(j, carry):
        i4 = j * 4
        uv = [(src_ref[0, 0, i4 + k], dst_ref[0, 0, i4 + k])
              for k in range(4)]
        masks = [(u, jnp.where(iota == (v >> 5), one << (v & 31), 0))
                 for (u, v) in uv]
        masks += [(v, jnp.where(iota == (u >> 5), one << (u & 31), 0))
                  for (u, v) in uv]
        for (r, m) in masks:
            adj_ref[pl.ds(r, 1), :] = adj_ref[pl.ds(r, 1), :] | m
        return carry

    jax.lax.fori_loop(0, n_chunk // 4, body, 0, unroll=4)


def _popcount(x):
    x = x - ((x >> 1) & 0x55555555)
    x = (x & 0x33333333) + ((x >> 2) & 0x33333333)
    x = (x + (x >> 4)) & 0x0F0F0F0F
    return (x * 0x01010101) >> 24


def _norm_kernel(src_ref, dst_ref, adj_ref, norm_ref, *, n_chunk):
    def body(j, carry):
        i8 = j * 8
        ru = jnp.concatenate(
            [adj_ref[pl.ds(src_ref[0, 0, i8 + k], 1), :] for k in range(8)],
            axis=0)
        rv = jnp.concatenate(
            [adj_ref[pl.ds(dst_ref[0, 0, i8 + k], 1), :] for k in range(8)],
            axis=0)
        inter = jnp.sum(_popcount(ru & rv), axis=1,
                        keepdims=True).astype(jnp.float32)
        union = jnp.sum(_popcount(ru | rv), axis=1,
                        keepdims=True).astype(jnp.float32)
        norm_ref[pl.ds(i8, 8), :] = inter / union
        return carry

    jax.lax.fori_loop(0, n_chunk // 8, body, 0, unroll=4)


def _matmul_kernel(x_ref, w_ref, o_ref):
    o_ref[...] = jnp.dot(x_ref[...], w_ref[...],
                         preferred_element_type=jnp.float32)


def _agg_kernel(src_ref, dst_ref, norm_ref, hl_ref, b_ref, out_ref, agg_ref,
                *, n_chunk):
    step = pl.program_id(0)
    nsteps = pl.num_programs(0)

    @pl.when(step == 0)
    def _():
        agg_ref[...] = jnp.full_like(agg_ref, -jnp.inf)

    def body(j, carry):
        i8 = j * 8
        rows = jnp.concatenate(
            [hl_ref[pl.ds(src_ref[0, 0, i8 + k], 1), :] for k in range(8)],
            axis=0)
        msgs = rows * norm_ref[pl.ds(i8, 8), :]
        for k in range(8):
            v = dst_ref[0, 0, i8 + k]
            agg_ref[pl.ds(v, 1), :] = jnp.maximum(agg_ref[pl.ds(v, 1), :],
                                                  msgs[k:k + 1, :])
        return carry

    jax.lax.fori_loop(0, n_chunk // 8, body, 0, unroll=4)

    @pl.when(step == nsteps - 1)
    def _():
        out_ref[...] = jnp.maximum(agg_ref[...] + b_ref[...], 0.0)


def _head_kernel(h_ref, w_ref, b_ref, o_ref):
    z = jnp.dot(h_ref[...], w_ref[...],
                preferred_element_type=jnp.float32) + b_ref[...]
    m = jnp.max(z, axis=-1, keepdims=True)
    lse = m + jnp.log(jnp.sum(jnp.exp(z - m), axis=-1, keepdims=True))
    o_ref[...] = z - lse


def kernel(x, edge_index, W0, b0, W1, b1, Wl, bl):
    n, d_in = x.shape
    d_hid = W0.shape[1]
    n_cls = Wl.shape[1]
    e = edge_index.shape[1]

    loops = jnp.arange(n, dtype=edge_index.dtype)
    ei = jnp.concatenate([edge_index, jnp.stack([loops, loops])], axis=1)
    e_tot = e + n
    ch = 4096
    nb = -(-e_tot // ch)
    e_pad = nb * ch
    ei = jnp.pad(ei, ((0, 0), (0, e_pad - e_tot)))
    src3 = ei[0].reshape(nb, 1, ch)
    dst3 = ei[1].reshape(nb, 1, ch)

    n_words = (-(-n // 32) + 127) // 128 * 128

    idx_spec = pl.BlockSpec((1, 1, ch), lambda i: (i, 0, 0),
                            memory_space=pltpu.SMEM)

    adj = pl.pallas_call(
        functools.partial(_build_adj_kernel, n_chunk=ch, n_words=n_words),
        grid=(nb,),
        in_specs=[idx_spec, idx_spec],
        out_specs=pl.BlockSpec((n, n_words), lambda i: (0, 0)),
        out_shape=jax.ShapeDtypeStruct((n, n_words), jnp.int32),
    )(src3, dst3)

    norm = pl.pallas_call(
        functools.partial(_norm_kernel, n_chunk=ch),
        grid=(nb,),
        in_specs=[idx_spec, idx_spec,
                  pl.BlockSpec((n, n_words), lambda i: (0, 0))],
        out_specs=pl.BlockSpec((ch, 1), lambda i: (i, 0)),
        out_shape=jax.ShapeDtypeStruct((e_pad, 1), jnp.float32),
    )(src3, dst3, adj)

    row_blk = 1000 if n % 1000 == 0 else n
    n_row_blks = n // row_blk

    def matmul(h, w):
        d_out = w.shape[1]
        return pl.pallas_call(
            _matmul_kernel,
            grid=(n_row_blks,),
            in_specs=[pl.BlockSpec((row_blk, w.shape[0]), lambda i: (i, 0)),
                      pl.BlockSpec(w.shape, lambda i: (0, 0))],
            out_specs=pl.BlockSpec((row_blk, d_out), lambda i: (i, 0)),
            out_shape=jax.ShapeDtypeStruct((n, d_out), jnp.float32),
        )(h, w)

    def layer(h, w, b):
        hl = matmul(h, w)
        return pl.pallas_call(
            functools.partial(_agg_kernel, n_chunk=ch),
            grid=(nb,),
            in_specs=[idx_spec, idx_spec,
                      pl.BlockSpec((ch, 1), lambda i: (i, 0)),
                      pl.BlockSpec((n, d_hid), lambda i: (0, 0)),
                      pl.BlockSpec((1, d_hid), lambda i: (0, 0))],
            out_specs=pl.BlockSpec((n, d_hid), lambda i: (0, 0)),
            out_shape=jax.ShapeDtypeStruct((n, d_hid), jnp.float32),
            scratch_shapes=[pltpu.VMEM((n, d_hid), jnp.float32)],
        )(src3, dst3, norm, hl, b.reshape(1, d_hid))

    h = layer(x, W0, b0)
    h = layer(h, W1, b1)

    return pl.pallas_call(
        _head_kernel,
        grid=(n_row_blks,),
        in_specs=[pl.BlockSpec((row_blk, d_hid), lambda i: (i, 0)),
                  pl.BlockSpec((d_hid, n_cls), lambda i: (0, 0)),
                  pl.BlockSpec((1, n_cls), lambda i: (0, 0))],
        out_specs=pl.BlockSpec((row_blk, n_cls), lambda i: (i, 0)),
        out_shape=jax.ShapeDtypeStruct((n, n_cls), jnp.float32),
    )(h, Wl, bl.reshape(1, n_cls))


# speedup vs baseline: 12.9807x; 1.1090x over previous
import functools

import jax
import jax.numpy as jnp
from jax.experimental import pallas as pl
from jax.experimental.pallas import tpu as pltpu


def _build_adj_kernel(src_ref, dst_ref, adj_ref, *, n_chunk, n_words):
    step = pl.program_id(0)

    @pl.when(step == 0)
    def _():
        adj_ref[...] = jnp.zeros_like(adj_ref)

    iota = jax.lax.broadcasted_iota(jnp.int32, (1, n_words), 1)
    one = jnp.int32(1)

    def body(j, carry):
        i4 = j * 4
        uv = [(src_ref[0, 0, i4 + k], dst_ref[0, 0, i4 + k])
              for k in range(4)]
        masks = [(u, jnp.where(iota == (v >> 5), one << (v & 31), 0))
                 for (u, v) in uv]
        masks += [(v, jnp.where(iota == (u >> 5), one << (u & 31), 0))
                  for (u, v) in uv]
        for (r, m) in masks:
            adj_ref[pl.ds(r, 1), :] = adj_ref[pl.ds(r, 1), :] | m
        return carry

    jax.lax.fori_loop(0, n_chunk // 4, body, 0, unroll=8)


def _popcount(x):
    x = x - ((x >> 1) & 0x55555555)
    x = (x & 0x33333333) + ((x >> 2) & 0x33333333)
    x = (x + (x >> 4)) & 0x0F0F0F0F
    return (x * 0x01010101) >> 24


def _norm_kernel(src_ref, dst_ref, adj_ref, norm_ref, *, n_chunk):
    def body(j, carry):
        i8 = j * 8
        ru = jnp.concatenate(
            [adj_ref[pl.ds(src_ref[0, 0, i8 + k], 1), :] for k in range(8)],
            axis=0)
        rv = jnp.concatenate(
            [adj_ref[pl.ds(dst_ref[0, 0, i8 + k], 1), :] for k in range(8)],
            axis=0)
        inter = jnp.sum(_popcount(ru & rv), axis=1,
                        keepdims=True).astype(jnp.float32)
        union = jnp.sum(_popcount(ru | rv), axis=1,
                        keepdims=True).astype(jnp.float32)
        norm_ref[pl.ds(i8, 8), :] = inter / union
        return carry

    jax.lax.fori_loop(0, n_chunk // 8, body, 0, unroll=8)


def _matmul_kernel(x_ref, w_ref, o_ref):
    o_ref[...] = jnp.dot(x_ref[...], w_ref[...],
                         preferred_element_type=jnp.float32)


def _agg_kernel(src_ref, dst_ref, norm_ref, hl_ref, b_ref, out_ref, agg_ref,
                *, n_chunk):
    step = pl.program_id(0)
    nsteps = pl.num_programs(0)

    @pl.when(step == 0)
    def _():
        agg_ref[...] = jnp.full_like(agg_ref, -jnp.inf)

    def body(j, carry):
        i8 = j * 8
        rows = jnp.concatenate(
            [hl_ref[pl.ds(src_ref[0, 0, i8 + k], 1), :] for k in range(8)],
            axis=0)
        msgs = rows * norm_ref[pl.ds(i8, 8), :]
        for k in range(8):
            v = dst_ref[0, 0, i8 + k]
            agg_ref[pl.ds(v, 1), :] = jnp.maximum(agg_ref[pl.ds(v, 1), :],
                                                  msgs[k:k + 1, :])
        return carry

    jax.lax.fori_loop(0, n_chunk // 8, body, 0, unroll=8)

    @pl.when(step == nsteps - 1)
    def _():
        out_ref[...] = jnp.maximum(agg_ref[...] + b_ref[...], 0.0)


def _head_kernel(h_ref, w_ref, b_ref, o_ref):
    z = jnp.dot(h_ref[...], w_ref[...],
                preferred_element_type=jnp.float32) + b_ref[...]
    m = jnp.max(z, axis=-1, keepdims=True)
    lse = m + jnp.log(jnp.sum(jnp.exp(z - m), axis=-1, keepdims=True))
    o_ref[...] = z - lse


def kernel(x, edge_index, W0, b0, W1, b1, Wl, bl):
    n, d_in = x.shape
    d_hid = W0.shape[1]
    n_cls = Wl.shape[1]
    e = edge_index.shape[1]

    loops = jnp.arange(n, dtype=edge_index.dtype)
    ei = jnp.concatenate([edge_index, jnp.stack([loops, loops])], axis=1)
    e_tot = e + n
    ch = 4096
    nb = -(-e_tot // ch)
    e_pad = nb * ch
    ei = jnp.pad(ei, ((0, 0), (0, e_pad - e_tot)))
    src3 = ei[0].reshape(nb, 1, ch)
    dst3 = ei[1].reshape(nb, 1, ch)

    n_words = (-(-n // 32) + 127) // 128 * 128

    idx_spec = pl.BlockSpec((1, 1, ch), lambda i: (i, 0, 0),
                            memory_space=pltpu.SMEM)

    adj = pl.pallas_call(
        functools.partial(_build_adj_kernel, n_chunk=ch, n_words=n_words),
        grid=(nb,),
        in_specs=[idx_spec, idx_spec],
        out_specs=pl.BlockSpec((n, n_words), lambda i: (0, 0)),
        out_shape=jax.ShapeDtypeStruct((n, n_words), jnp.int32),
    )(src3, dst3)

    norm = pl.pallas_call(
        functools.partial(_norm_kernel, n_chunk=ch),
        grid=(nb,),
        in_specs=[idx_spec, idx_spec,
                  pl.BlockSpec((n, n_words), lambda i: (0, 0))],
        out_specs=pl.BlockSpec((ch, 1), lambda i: (i, 0)),
        out_shape=jax.ShapeDtypeStruct((e_pad, 1), jnp.float32),
    )(src3, dst3, adj)

    row_blk = 1000 if n % 1000 == 0 else n
    n_row_blks = n // row_blk

    def matmul(h, w):
        d_out = w.shape[1]
        return pl.pallas_call(
            _matmul_kernel,
            grid=(n_row_blks,),
            in_specs=[pl.BlockSpec((row_blk, w.shape[0]), lambda i: (i, 0)),
                      pl.BlockSpec(w.shape, lambda i: (0, 0))],
            out_specs=pl.BlockSpec((row_blk, d_out), lambda i: (i, 0)),
            out_shape=jax.ShapeDtypeStruct((n, d_out), jnp.float32),
        )(h, w)

    def layer(h, w, b):
        hl = matmul(h, w)
        return pl.pallas_call(
            functools.partial(_agg_kernel, n_chunk=ch),
            grid=(nb,),
            in_specs=[idx_spec, idx_spec,
                      pl.BlockSpec((ch, 1), lambda i: (i, 0)),
                      pl.BlockSpec((n, d_hid), lambda i: (0, 0)),
                      pl.BlockSpec((1, d_hid), lambda i: (0, 0))],
            out_specs=pl.BlockSpec((n, d_hid), lambda i: (0, 0)),
            out_shape=jax.ShapeDtypeStruct((n, d_hid), jnp.float32),
            scratch_shapes=[pltpu.VMEM((n, d_hid), jnp.float32)],
        )(src3, dst3, norm, hl, b.reshape(1, d_hid))

    h = layer(x, W0, b0)
    h = layer(h, W1, b1)

    return pl.pallas_call(
        _head_kernel,
        grid=(n_row_blks,),
        in_specs=[pl.BlockSpec((row_blk, d_hid), lambda i: (i, 0)),
                  pl.BlockSpec((d_hid, n_cls), lambda i: (0, 0)),
                  pl.BlockSpec((1, n_cls), lambda i: (0, 0))],
        out_specs=pl.BlockSpec((row_blk, n_cls), lambda i: (i, 0)),
        out_shape=jax.ShapeDtypeStruct((n, n_cls), jnp.float32),
    )(h, Wl, bl.reshape(1, n_cls))
